# Initial kernel scaffold; baseline (speedup 1.0000x reference)
import jax
import jax.numpy as jnp
from jax.experimental import pallas as pl


def kernel(x, edge_index, W1, a_src1, a_dst1, b1, W2, a_src2, a_dst2, b2):
    raise NotImplementedError("write your pallas kernel here")


# speedup vs baseline: 37.1847x; 37.1847x over previous
import functools

import jax
import jax.numpy as jnp
from jax import lax
from jax.experimental import pallas as pl
from jax.experimental.pallas import tpu as pltpu
from jax.experimental.pallas import tpu_sc as plsc

N = 10000
E = 320000
DIN = 128
HID = 64
H1 = 5
C1 = H1 * HID
CH = C1 // 2
NC = 2
NS = 16
L = 16
SR = N // NS
B = 80
NB1 = E // NS // B
NB2 = E // (NC * NS) // B
NBLK = 10
NR = N // NBLK

@functools.cache
def _mesh():
    return plsc.VectorSubcoreMesh(core_axis_name="c", subcore_axis_name="s",
                                  num_cores=NC, num_subcores=NS)


def _leaky(x):
    return jnp.where(x >= 0, x, 0.2 * x)


def _tc1_body(x_ref, w1_ref, as_ref, ad_ref, h1t_ref, s_ref, d_ref):
    h = jnp.dot(x_ref[...], w1_ref[...], preferred_element_type=jnp.float32)
    h1t_ref[0] = h[:, :CH]
    h1t_ref[1] = h[:, CH:]
    s_ref[...] = jnp.dot(h, as_ref[...], preferred_element_type=jnp.float32)
    d_ref[...] = jnp.dot(h, ad_ref[...], preferred_element_type=jnp.float32)


def _tc1(x, W1, As16, Ad16):
    return pl.pallas_call(
        _tc1_body,
        grid=(NBLK,),
        in_specs=[
            pl.BlockSpec((NR, DIN), lambda i: (i, 0)),
            pl.BlockSpec((DIN, C1), lambda i: (0, 0)),
            pl.BlockSpec((C1, 16), lambda i: (0, 0)),
            pl.BlockSpec((C1, 16), lambda i: (0, 0)),
        ],
        out_specs=[
            pl.BlockSpec((2, NR, CH), lambda i: (0, i, 0)),
            pl.BlockSpec((NR, 16), lambda i: (i, 0)),
            pl.BlockSpec((NR, 16), lambda i: (i, 0)),
        ],
        out_shape=[
            jax.ShapeDtypeStruct((2, N, CH), jnp.float32),
            jax.ShapeDtypeStruct((N, 16), jnp.float32),
            jax.ShapeDtypeStruct((N, 16), jnp.float32),
        ],
    )(x, W1, As16, Ad16)


def _zero_vmem(ref, nrows, ncols):
    z = jnp.zeros((L,), jnp.float32)

    def row(r, _):
        for v in range(ncols // L):
            ref[r, pl.ds(v * L, L)] = z
        return 0

    lax.fori_loop(0, nrows, row, 0)


def _sc1_block(off, base, src_hbm, dst_hbm, h1t_hbm, s_tab, d_tab,
               src_v, dst_v, adj_v, s_rows, d_rows, ex_v, rows_v,
               sems, do_den, den_sh, accum_sh, ncols, hv_list):
    nv = ncols // L
    pltpu.sync_copy(src_hbm.at[pl.ds(base, B)], src_v)
    pltpu.sync_copy(dst_hbm.at[pl.ds(base, B)], dst_v)
    if off:
        for i in range(B // L):
            adj_v[pl.ds(i * L, L)] = src_v[pl.ds(i * L, L)] + off
        idx_ref = adj_v
    else:
        idx_ref = src_v
    g0 = pltpu.async_copy(s_tab.at[src_v], s_rows, sems[0])
    g1 = pltpu.async_copy(d_tab.at[dst_v], d_rows, sems[1])
    g2 = pltpu.async_copy(h1t_hbm.at[idx_ref], rows_v, sems[2])
    g0.wait()
    g1.wait()

    def exbody(e, _):
        a = s_rows[e] + d_rows[e]
        ex_v[e] = jnp.exp(_leaky(a))
        return 0

    lax.fori_loop(0, B, exbody, 0)
    if do_den:
        pltpu.sync_copy(ex_v, den_sh.at[dst_v], add=True)
    g2.wait()

    def mulbody(e, _):
        ex_row = ex_v[e]
        for v in range(nv):
            ev = ex_row[hv_list[v]]
            rows_v[e, pl.ds(v * L, L)] = rows_v[e, pl.ds(v * L, L)] * ev
        return 0

    lax.fori_loop(0, B, mulbody, 0)
    pltpu.sync_copy(rows_v, accum_sh.at[dst_v], add=True)


def _zero_stripe(rows_v, ex_v, accum_sh, den_sh, s, ncols):
    _zero_vmem(rows_v, B, ncols)
    _zero_vmem(ex_v, B, 16)
    for j in range(SR // B):
        pltpu.sync_copy(rows_v, accum_sh.at[pl.ds(s * SR + j * B, B)])
        pltpu.sync_copy(ex_v, den_sh.at[pl.ds(s * SR + j * B, B)])
    rem = SR % B
    base = s * SR + (SR // B) * B
    pltpu.sync_copy(rows_v.at[pl.ds(0, rem)], accum_sh.at[pl.ds(base, rem)])
    pltpu.sync_copy(ex_v.at[pl.ds(0, rem)], den_sh.at[pl.ds(base, rem)])


def _sc1_kernel(src_hbm, dst_hbm, h1t_hbm, s_tab, d_tab,
                num_hbm, den_hbm,
                accum_sh, den_sh,
                src_v, dst_v, adj_v, s_rows, d_rows, ex_v, rows_v,
                sem0, sem1, sem2):
    c = lax.axis_index("c")
    s = lax.axis_index("s")
    _zero_stripe(rows_v, ex_v, accum_sh, den_sh, s, CH)
    plsc.subcore_barrier()

    sems = (sem0, sem1, sem2)

    def run(off, hv_list, do_den):
        def blk(b, _):
            base = s * (E // NS) + b * B
            _sc1_block(off, base, src_hbm, dst_hbm, h1t_hbm, s_tab, d_tab,
                       src_v, dst_v, adj_v, s_rows, d_rows, ex_v, rows_v,
                       sems, do_den, den_sh, accum_sh, CH, hv_list)
            return 0

        lax.fori_loop(0, NB1, blk, 0)

    @pl.when(c == 0)
    def _():
        run(0, [v // 4 for v in range(CH // L)], True)

    @pl.when(c == 1)
    def _():
        run(N, [(10 + v) // 4 for v in range(CH // L)], False)
    plsc.subcore_barrier()
    row0 = c * N + s * SR
    pltpu.sync_copy(accum_sh.at[pl.ds(s * SR, SR)], num_hbm.at[pl.ds(row0, SR)])

    @pl.when(c == 0)
    def _():
        pltpu.sync_copy(den_sh.at[pl.ds(s * SR, SR)], den_hbm.at[pl.ds(s * SR, SR)])


def _sc1(src, dst, h1t_flat, s_tab, d_tab):
    f = functools.partial(
        pl.kernel,
        out_type=(
            jax.ShapeDtypeStruct((2 * N, CH), jnp.float32),
            jax.ShapeDtypeStruct((N, 16), jnp.float32),
        ),
        mesh=_mesh(),
        compiler_params=pltpu.CompilerParams(use_tc_tiling_on_sc=False),
        scratch_types=[
            pltpu.VMEM_SHARED((N, CH), jnp.float32),
            pltpu.VMEM_SHARED((N, 16), jnp.float32),
            pltpu.VMEM((B,), jnp.int32),
            pltpu.VMEM((B,), jnp.int32),
            pltpu.VMEM((B,), jnp.int32),
            pltpu.VMEM((B, 16), jnp.float32),
            pltpu.VMEM((B, 16), jnp.float32),
            pltpu.VMEM((B, 16), jnp.float32),
            pltpu.VMEM((B, CH), jnp.float32),
            pltpu.SemaphoreType.DMA,
            pltpu.SemaphoreType.DMA,
            pltpu.SemaphoreType.DMA,
        ],
    )(_sc1_kernel)
    return f(src, dst, h1t_flat, s_tab, d_tab)


def _sc2_kernel(src_hbm, dst_hbm, h2t_hbm, s2_tab, d2_tab,
                num_hbm, den_hbm,
                accum_sh, den_sh,
                src_v, dst_v, adj_v, s_rows, d_rows, ex_v, rows_v,
                sem0, sem1, sem2):
    c = lax.axis_index("c")
    s = lax.axis_index("s")
    _zero_stripe(rows_v, ex_v, accum_sh, den_sh, s, HID)
    plsc.subcore_barrier()

    sems = (sem0, sem1, sem2)
    wid = s * NC + c

    def blk(b, _):
        base = wid * (E // (NC * NS)) + b * B
        _sc1_block(0, base, src_hbm, dst_hbm, h2t_hbm, s2_tab, d2_tab,
                   src_v, dst_v, adj_v, s_rows, d_rows, ex_v, rows_v,
                   sems, True, den_sh, accum_sh, HID, [0] * (HID // L))
        return 0

    lax.fori_loop(0, NB2, blk, 0)
    plsc.subcore_barrier()
    row0 = c * N + s * SR
    pltpu.sync_copy(accum_sh.at[pl.ds(s * SR, SR)], num_hbm.at[pl.ds(row0, SR)])
    pltpu.sync_copy(den_sh.at[pl.ds(s * SR, SR)], den_hbm.at[pl.ds(row0, SR)])


def _sc2(src, dst, h2t, s2_tab, d2_tab):
    f = functools.partial(
        pl.kernel,
        out_type=(
            jax.ShapeDtypeStruct((2 * N, HID), jnp.float32),
            jax.ShapeDtypeStruct((2 * N, 16), jnp.float32),
        ),
        mesh=_mesh(),
        compiler_params=pltpu.CompilerParams(use_tc_tiling_on_sc=False),
        scratch_types=[
            pltpu.VMEM_SHARED((N, HID), jnp.float32),
            pltpu.VMEM_SHARED((N, 16), jnp.float32),
            pltpu.VMEM((B,), jnp.int32),
            pltpu.VMEM((B,), jnp.int32),
            pltpu.VMEM((B,), jnp.int32),
            pltpu.VMEM((B, 16), jnp.float32),
            pltpu.VMEM((B, 16), jnp.float32),
            pltpu.VMEM((B, 16), jnp.float32),
            pltpu.VMEM((B, HID), jnp.float32),
            pltpu.SemaphoreType.DMA,
            pltpu.SemaphoreType.DMA,
            pltpu.SemaphoreType.DMA,
        ],
    )(_sc2_kernel)
    return f(src, dst, h2t, s2_tab, d2_tab)


_SEQ_A = ((0, 64), (1, 64), (2, 32))
_SEQ_B = ((2, 32), (3, 64), (4, 64))


def _expand(vals, seq):
    return jnp.concatenate(
        [jnp.broadcast_to(vals[:, h:h + 1], (vals.shape[0], w)) for h, w in seq],
        axis=1)


def _tc2_body(na_ref, nb_ref, den_ref, s_ref, d_ref, ha_ref, hb_ref,
              w2_ref, as2_ref, ad2_ref, b1_ref,
              h2_ref, s2_ref, d2_ref):
    alpha = s_ref[...] + d_ref[...]
    exs = jnp.exp(_leaky(alpha))
    denom = den_ref[...] + exs
    fa = (na_ref[...] + _expand(exs, _SEQ_A) * ha_ref[...]) / _expand(denom, _SEQ_A)
    fb = (nb_ref[...] + _expand(exs, _SEQ_B) * hb_ref[...]) / _expand(denom, _SEQ_B)
    hr = jnp.maximum(jnp.concatenate([fa, fb], axis=1) + b1_ref[...], 0.0)
    h2 = jnp.dot(hr, w2_ref[...], preferred_element_type=jnp.float32)
    h2_ref[...] = h2
    s2_ref[...] = jnp.dot(h2, as2_ref[...], preferred_element_type=jnp.float32)
    d2_ref[...] = jnp.dot(h2, ad2_ref[...], preferred_element_type=jnp.float32)


def _tc2(num, den, s_tab, d_tab, h1t_flat, W2, As2, Ad2, b1r):
    return pl.pallas_call(
        _tc2_body,
        grid=(NBLK,),
        in_specs=[
            pl.BlockSpec((NR, CH), lambda i: (i, 0)),
            pl.BlockSpec((NR, CH), lambda i: (NBLK + i, 0)),
            pl.BlockSpec((NR, 16), lambda i: (i, 0)),
            pl.BlockSpec((NR, 16), lambda i: (i, 0)),
            pl.BlockSpec((NR, 16), lambda i: (i, 0)),
            pl.BlockSpec((NR, CH), lambda i: (i, 0)),
            pl.BlockSpec((NR, CH), lambda i: (NBLK + i, 0)),
            pl.BlockSpec((C1, HID), lambda i: (0, 0)),
            pl.BlockSpec((HID, 16), lambda i: (0, 0)),
            pl.BlockSpec((HID, 16), lambda i: (0, 0)),
            pl.BlockSpec((1, C1), lambda i: (0, 0)),
        ],
        out_specs=[
            pl.BlockSpec((NR, HID), lambda i: (i, 0)),
            pl.BlockSpec((NR, 16), lambda i: (i, 0)),
            pl.BlockSpec((NR, 16), lambda i: (i, 0)),
        ],
        out_shape=[
            jax.ShapeDtypeStruct((N, HID), jnp.float32),
            jax.ShapeDtypeStruct((N, 16), jnp.float32),
            jax.ShapeDtypeStruct((N, 16), jnp.float32),
        ],
    )(num, num, den, s_tab, d_tab, h1t_flat, h1t_flat, W2, As2, Ad2, b1r)


def _tc3_body(na_ref, nb_ref, da_ref, db_ref, s2_ref, d2_ref, h2_ref,
              b2_ref, out_ref):
    alpha = s2_ref[...] + d2_ref[...]
    exs = jnp.exp(_leaky(alpha))
    e = exs[:, 0:1]
    numv = na_ref[...] + nb_ref[...] + e * h2_ref[...]
    denv = da_ref[:, 0:1] + db_ref[:, 0:1] + e
    out_ref[...] = jnp.maximum(numv / denv + b2_ref[...], 0.0)


def _tc3(num2, den2, s2_tab, d2_tab, h2t, b2r):
    return pl.pallas_call(
        _tc3_body,
        grid=(NBLK,),
        in_specs=[
            pl.BlockSpec((NR, HID), lambda i: (i, 0)),
            pl.BlockSpec((NR, HID), lambda i: (NBLK + i, 0)),
            pl.BlockSpec((NR, 16), lambda i: (i, 0)),
            pl.BlockSpec((NR, 16), lambda i: (NBLK + i, 0)),
            pl.BlockSpec((NR, 16), lambda i: (i, 0)),
            pl.BlockSpec((NR, 16), lambda i: (i, 0)),
            pl.BlockSpec((NR, HID), lambda i: (i, 0)),
            pl.BlockSpec((1, HID), lambda i: (0, 0)),
        ],
        out_specs=pl.BlockSpec((NR, HID), lambda i: (i, 0)),
        out_shape=jax.ShapeDtypeStruct((N, HID), jnp.float32),
    )(num2, num2, den2, den2, s2_tab, d2_tab, h2t, b2r)


def kernel(x, edge_index, W1, a_src1, a_dst1, b1, W2, a_src2, a_dst2, b2):
    src = edge_index[0]
    dst = edge_index[1]
    eye = jnp.eye(H1, dtype=jnp.float32)
    As = (eye[:, None, :] * a_src1[:, :, None]).reshape(C1, H1)
    Ad = (eye[:, None, :] * a_dst1[:, :, None]).reshape(C1, H1)
    As16 = jnp.pad(As, ((0, 0), (0, 16 - H1)))
    Ad16 = jnp.pad(Ad, ((0, 0), (0, 16 - H1)))
    As2 = jnp.pad(a_src2.T, ((0, 0), (0, 15)))
    Ad2 = jnp.pad(a_dst2.T, ((0, 0), (0, 15)))

    h1t, s_tab, d_tab = _tc1(x, W1, As16, Ad16)
    h1t_flat = h1t.reshape(2 * N, CH)
    num, den = _sc1(src, dst, h1t_flat, s_tab, d_tab)
    h2t, s2_tab, d2_tab = _tc2(num, den, s_tab, d_tab, h1t_flat,
                               W2, As2, Ad2, b1.reshape(1, C1))
    num2, den2 = _sc2(src, dst, h2t, s2_tab, d2_tab)
    return _tc3(num2, den2, s2_tab, d2_tab, h2t, b2.reshape(1, HID))
